# baseline (speedup 1.0000x reference)
import functools

import jax
import jax.numpy as jnp
from jax import lax
from jax.experimental import pallas as pl
from jax.experimental.pallas import tpu as pltpu
from jax.experimental.pallas import tpu_sc as plsc

_NC = 2
_NS = 16
_NW = _NC * _NS
_L = 16
_W = 2048
_TILE = 128
_ICHUNK = 128


def _detile_body(V, D, n_full, rem_w, tail_base, tail_n,
                 tbl_a, tbl_b, tail_a, tail_b, out_a, out_b,
                 buf0, buf1, tbuf, so0, so1):
    bufs = (buf0, buf1)
    sems = (so0, so1)
    wid = lax.axis_index("s") * _NC + lax.axis_index("c")
    n_rounds = -(-(n_full + 1) // _NW)

    D_ = len(bufs[0].shape) and bufs[0].shape[0]

    def fire(tbl, out, c, rb):
        @pl.when((c >= 0) & (c < n_full))
        def _full():
            start = pl.multiple_of(c * _W, _TILE)
            pltpu.sync_copy(tbl.at[:, pl.ds(start, _W)], bufs[rb])
            for d in range(D_):
                pltpu.async_copy(bufs[rb].at[d],
                                 out.at[d, pl.ds(start, _W)], sems[rb])

        @pl.when(c == n_full)
        def _rem():
            start = pl.multiple_of(n_full * _W, _TILE)
            pltpu.sync_copy(tbl.at[:, pl.ds(start, rem_w)],
                            bufs[rb].at[:, pl.ds(0, rem_w)])
            for d in range(D_):
                pltpu.async_copy(bufs[rb].at[d, pl.ds(0, rem_w)],
                                 out.at[d, pl.ds(start, rem_w)], sems[rb])

    def drain(out, c, rb):
        @pl.when((c >= 0) & (c < n_full))
        def _df():
            start = pl.multiple_of(c * _W, _TILE)
            for d in range(D_):
                pltpu.make_async_copy(
                    bufs[rb].at[d], out.at[d, pl.ds(start, _W)],
                    sems[rb]).wait()

        @pl.when(c == n_full)
        def _dr():
            start = pl.multiple_of(n_full * _W, _TILE)
            for d in range(D_):
                pltpu.make_async_copy(
                    bufs[rb].at[d, pl.ds(0, rem_w)],
                    out.at[d, pl.ds(start, rem_w)], sems[rb]).wait()

    n_pairs = -(-n_rounds // 2)
    for tbl, out in ((tbl_a, out_a), (tbl_b, out_b)):
        def pair(kk, _, tbl=tbl, out=out):
            k0 = 2 * kk
            drain(out, wid + (k0 - 2) * _NW, 0)
            fire(tbl, out, wid + k0 * _NW, 0)
            drain(out, wid + (k0 - 1) * _NW, 1)
            fire(tbl, out, wid + (k0 + 1) * _NW, 1)
            return _

        lax.fori_loop(0, n_pairs, pair, None)
        for k in (2 * n_pairs - 2, 2 * n_pairs - 1):
            drain(out, wid + k * _NW, k % 2)

    @pl.when(wid == _NW - 1)
    def _tails():
        for tail, out in ((tail_a, out_a), (tail_b, out_b)):
            pltpu.sync_copy(tail, tbuf)
            pltpu.sync_copy(tbuf, out.at[:, pl.ds(tail_base, tail_n)])


def _gather_body(n_per_w, D, idx0_hbm, idx1_hbm, idx2_hbm, lin_in_hbm,
                 lin_w_hbm, emb_b_hbm, out_hbm,
                 idx0_v, idx1_v, idx2_v, e0_v, e1_v, w_v, b_v, out_v, sem):
    wid = lax.axis_index("s") * _NC + lax.axis_index("c")
    base = wid * n_per_w

    pltpu.sync_copy(idx0_hbm.at[pl.ds(base, n_per_w)], idx0_v)
    pltpu.sync_copy(idx1_hbm.at[pl.ds(base, n_per_w)], idx1_v)
    pltpu.sync_copy(idx2_hbm.at[pl.ds(base, n_per_w)], idx2_v)

    copies = []
    for k in range(0, n_per_w, _ICHUNK):
        sl = pl.ds(k, _ICHUNK)
        for d in range(D):
            copies.append(pltpu.async_copy(
                lin_in_hbm.at[d].at[idx0_v.at[sl]], e0_v.at[d].at[sl], sem))
            copies.append(pltpu.async_copy(
                lin_in_hbm.at[d].at[idx1_v.at[sl]], e1_v.at[d].at[sl], sem))
            copies.append(pltpu.async_copy(
                lin_w_hbm.at[d].at[idx2_v.at[sl]], w_v.at[d].at[sl], sem))
        copies.append(pltpu.async_copy(
            emb_b_hbm.at[idx2_v.at[sl]], b_v.at[sl], sem))
    for c in copies:
        c.wait()

    def group(g, _):
        sl = pl.ds(g * _L, _L)
        acc = jnp.zeros((_L,), jnp.float32)
        for d in range(D):
            acc = acc + (e0_v[d, sl] + e1_v[d, sl]) * w_v[d, sl]
        logit = acc * 0.5 + b_v[sl]
        out_v[sl] = 1.0 / (1.0 + jnp.exp(-logit))
        return _

    lax.fori_loop(0, n_per_w // _L, group, None)

    pltpu.sync_copy(out_v, out_hbm.at[pl.ds(base, n_per_w)])


def kernel(x, emb_in, emb_out_w, emb_out_b):
    B = x.shape[0]
    V, D = emb_in.shape
    tail_base = (V // _TILE) * _TILE
    tail_n = V - tail_base
    n_full = V // _W
    rem_w = tail_base - n_full * _W
    n_per_w = B // _NW

    mesh = plsc.VectorSubcoreMesh(core_axis_name="c", subcore_axis_name="s")

    detile = pl.kernel(
        functools.partial(_detile_body, V, D, n_full, rem_w, tail_base,
                          tail_n),
        out_type=(jax.ShapeDtypeStruct((D, V), jnp.float32),
                  jax.ShapeDtypeStruct((D, V), jnp.float32)),
        mesh=mesh,
        scratch_types=[
            pltpu.VMEM((D, _W), jnp.float32),
            pltpu.VMEM((D, _W), jnp.float32),
            pltpu.VMEM((D, 64), jnp.float32),
            pltpu.SemaphoreType.DMA,
            pltpu.SemaphoreType.DMA,
        ],
        compiler_params=pltpu.CompilerParams(
            needs_layout_passes=False, use_tc_tiling_on_sc=True),
    )
    tail_in = emb_in[tail_base:].T
    tail_w = emb_out_w[tail_base:].T
    lin_in, lin_w = detile(emb_in.T, emb_out_w.T, tail_in, tail_w)

    gather = pl.kernel(
        functools.partial(_gather_body, n_per_w, D),
        out_type=jax.ShapeDtypeStruct((B,), jnp.float32),
        mesh=mesh,
        scratch_types=[
            pltpu.VMEM((n_per_w,), jnp.int32),
            pltpu.VMEM((n_per_w,), jnp.int32),
            pltpu.VMEM((n_per_w,), jnp.int32),
            pltpu.VMEM((D, n_per_w), jnp.float32),
            pltpu.VMEM((D, n_per_w), jnp.float32),
            pltpu.VMEM((D, n_per_w), jnp.float32),
            pltpu.VMEM((n_per_w,), jnp.float32),
            pltpu.VMEM((n_per_w,), jnp.float32),
            pltpu.SemaphoreType.DMA,
        ],
        compiler_params=pltpu.CompilerParams(
            needs_layout_passes=False, use_tc_tiling_on_sc=False),
    )
    out = gather(x[:, 0], x[:, 1], x[:, 2], lin_in, lin_w,
                 emb_out_b.reshape(V))
    return out.reshape(B, 1)


# speedup vs baseline: 2.7965x; 2.7965x over previous
import functools

import jax
import jax.numpy as jnp
from jax import lax
from jax.experimental import pallas as pl
from jax.experimental.pallas import tpu as pltpu
from jax.experimental.pallas import tpu_sc as plsc

_NC = 2
_NS = 16
_NW = _NC * _NS
_L = 16
_TILE = 128
_ICHUNK = 128


def _detile_body(V, D, n_ch, tail_base, tail_n,
                 tbl_a, tbl_b, tail_a, tail_b, out_a, out_b,
                 blk0, blk1, tv0, tv1, tbuf, s0, s1, so0, so1):
    blks = (blk0, blk1)
    tvs = (tv0, tv1)
    isems = (s0, s1)
    osems = (so0, so1)
    wid = lax.axis_index("s") * _NC + lax.axis_index("c")
    lanes = lax.iota(jnp.int32, _L)
    n_rounds = -(-n_ch // _NW)

    def fire_in(tbl, c, rb):
        @pl.when(c < n_ch)
        def _f():
            start = pl.multiple_of(c * _TILE, _TILE)
            pltpu.async_copy(tbl.at[0, :, pl.ds(start, _TILE)],
                             blks[rb].at[0], isems[rb])
            pltpu.async_copy(tbl.at[1, :, pl.ds(start, _TILE)],
                             blks[rb].at[1], isems[rb])

    def process(out, c, rb):
        @pl.when(c < n_ch)
        def _p():
            pltpu.make_async_copy(
                tbl_a.at[0, :, pl.ds(0, _TILE)], blks[rb].at[0],
                isems[rb]).wait()
            pltpu.make_async_copy(
                tbl_a.at[0, :, pl.ds(0, _TILE)], blks[rb].at[1],
                isems[rb]).wait()
            hv = lanes >> 3
            sv = lanes & 7
            for j in range(_TILE):
                vals = plsc.load_gather(
                    blks[rb], [hv, sv, jnp.full((_L,), j, jnp.int32)])
                tvs[rb][pl.ds(j * D, D)] = vals
            start = pl.multiple_of(c * _TILE * D, _TILE)
            pltpu.async_copy(tvs[rb], out.at[pl.ds(start, _TILE * D)],
                             osems[rb])

    def drain_out(out, c, rb):
        @pl.when((c >= 0) & (c < n_ch))
        def _d():
            start = pl.multiple_of(c * _TILE * D, _TILE)
            pltpu.make_async_copy(
                tvs[rb], out.at[pl.ds(start, _TILE * D)], osems[rb]).wait()

    for tbl, out in ((tbl_a, out_a), (tbl_b, out_b)):
        def pair(kk, _, tbl=tbl, out=out):
            k0 = 2 * kk
            c0 = wid + k0 * _NW
            c1 = wid + (k0 + 1) * _NW
            drain_out(out, wid + (k0 - 2) * _NW, 0)
            fire_in(tbl, c0, 0)
            drain_out(out, wid + (k0 - 1) * _NW, 1)
            fire_in(tbl, c1, 1)
            process(out, c0, 0)
            process(out, c1, 1)
            return _

        lax.fori_loop(0, -(-n_rounds // 2), pair, None)
        np2 = 2 * (-(-n_rounds // 2))
        for k in (np2 - 2, np2 - 1):
            drain_out(out, wid + k * _NW, k % 2)

    @pl.when(wid == _NW - 1)
    def _tails():
        for tail, out in ((tail_a, out_a), (tail_b, out_b)):
            pltpu.sync_copy(tail, tbuf)
            pltpu.sync_copy(tbuf, out.at[pl.ds(tail_base * D, tail_n * D)])


def _gather_body(n_per_w, D, idx0_hbm, idx1_hbm, idx2_hbm, emb_in_hbm,
                 emb_w_hbm, emb_b_hbm, out_hbm,
                 idx0_v, idx1_v, idx2_v, e0_v, e1_v, w_v, b_v, out_v, sem):
    wid = lax.axis_index("s") * _NC + lax.axis_index("c")
    base = wid * n_per_w

    pltpu.sync_copy(idx0_hbm.at[pl.ds(base, n_per_w)], idx0_v)
    pltpu.sync_copy(idx1_hbm.at[pl.ds(base, n_per_w)], idx1_v)
    pltpu.sync_copy(idx2_hbm.at[pl.ds(base, n_per_w)], idx2_v)

    copies = []
    for k in range(0, n_per_w, _ICHUNK):
        sl = pl.ds(k, _ICHUNK)
        copies.append(pltpu.async_copy(
            emb_in_hbm.at[idx0_v.at[sl]], e0_v.at[sl], sem))
        copies.append(pltpu.async_copy(
            emb_in_hbm.at[idx1_v.at[sl]], e1_v.at[sl], sem))
        copies.append(pltpu.async_copy(
            emb_w_hbm.at[idx2_v.at[sl]], w_v.at[sl], sem))
        copies.append(pltpu.async_copy(
            emb_b_hbm.at[idx2_v.at[sl]], b_v.at[sl], sem))
    for c in copies:
        c.wait()

    lanes = lax.iota(jnp.int32, _L)

    def group(g, _):
        rows = g * _L + lanes
        acc = jnp.zeros((_L,), jnp.float32)
        for d in range(D):
            col = jnp.full((_L,), d, jnp.int32)
            a0 = plsc.load_gather(e0_v, [rows, col])
            a1 = plsc.load_gather(e1_v, [rows, col])
            aw = plsc.load_gather(w_v, [rows, col])
            acc = acc + (a0 + a1) * aw
        logit = acc * 0.5 + b_v[pl.ds(g * _L, _L)]
        out_v[pl.ds(g * _L, _L)] = 1.0 / (1.0 + jnp.exp(-logit))
        return _

    lax.fori_loop(0, n_per_w // _L, group, None)

    pltpu.sync_copy(out_v, out_hbm.at[pl.ds(base, n_per_w)])


def kernel(x, emb_in, emb_out_w, emb_out_b):
    B = x.shape[0]
    V, D = emb_in.shape
    tail_base = (V // _TILE) * _TILE
    tail_n = V - tail_base
    n_ch = V // _TILE
    n_per_w = B // _NW

    mesh = plsc.VectorSubcoreMesh(core_axis_name="c", subcore_axis_name="s")

    detile = pl.kernel(
        functools.partial(_detile_body, V, D, n_ch, tail_base, tail_n),
        out_type=(jax.ShapeDtypeStruct((V * D,), jnp.float32),
                  jax.ShapeDtypeStruct((V * D,), jnp.float32)),
        mesh=mesh,
        scratch_types=[
            pltpu.VMEM((2, 8, _TILE), jnp.float32),
            pltpu.VMEM((2, 8, _TILE), jnp.float32),
            pltpu.VMEM((_TILE * D,), jnp.float32),
            pltpu.VMEM((_TILE * D,), jnp.float32),
            pltpu.VMEM((tail_n * D,), jnp.float32),
            pltpu.SemaphoreType.DMA,
            pltpu.SemaphoreType.DMA,
            pltpu.SemaphoreType.DMA,
            pltpu.SemaphoreType.DMA,
        ],
        compiler_params=pltpu.CompilerParams(
            needs_layout_passes=False, use_tc_tiling_on_sc=True),
    )
    tbl_in = emb_in.T.reshape(2, 8, V)
    tbl_w = emb_out_w.T.reshape(2, 8, V)
    tail_in = emb_in[tail_base:].reshape(-1)
    tail_w = emb_out_w[tail_base:].reshape(-1)
    lin_in_flat, lin_w_flat = detile(tbl_in, tbl_w, tail_in, tail_w)
    lin_in = lin_in_flat.reshape(V, D)
    lin_w = lin_w_flat.reshape(V, D)

    gather = pl.kernel(
        functools.partial(_gather_body, n_per_w, D),
        out_type=jax.ShapeDtypeStruct((B,), jnp.float32),
        mesh=mesh,
        scratch_types=[
            pltpu.VMEM((n_per_w,), jnp.int32),
            pltpu.VMEM((n_per_w,), jnp.int32),
            pltpu.VMEM((n_per_w,), jnp.int32),
            pltpu.VMEM((n_per_w, D), jnp.float32),
            pltpu.VMEM((n_per_w, D), jnp.float32),
            pltpu.VMEM((n_per_w, D), jnp.float32),
            pltpu.VMEM((n_per_w,), jnp.float32),
            pltpu.VMEM((n_per_w,), jnp.float32),
            pltpu.SemaphoreType.DMA,
        ],
        compiler_params=pltpu.CompilerParams(
            needs_layout_passes=False, use_tc_tiling_on_sc=False),
    )
    out = gather(x[:, 0], x[:, 1], x[:, 2], lin_in, lin_w,
                 emb_out_b.reshape(V))
    return out.reshape(B, 1)


# speedup vs baseline: 3.3294x; 1.1906x over previous
import functools

import jax
import jax.numpy as jnp
from jax import lax
from jax.experimental import pallas as pl
from jax.experimental.pallas import tpu as pltpu
from jax.experimental.pallas import tpu_sc as plsc

_NC = 2
_NS = 16
_NW = _NC * _NS
_L = 16
_CHUNK = 128


def _cbow_body(n_per_w, D, idx0_hbm, idx1_hbm, idx2_hbm, emb_in_hbm,
               emb_w_hbm, emb_b_hbm, out_hbm,
               idx0_v, idx1_v, idx2_v, e0_v, e1_v, w_v, b_v, out_v, sem):
    wid = lax.axis_index("s") * _NC + lax.axis_index("c")
    base = wid * n_per_w

    pltpu.sync_copy(idx0_hbm.at[pl.ds(base, n_per_w)], idx0_v)
    pltpu.sync_copy(idx1_hbm.at[pl.ds(base, n_per_w)], idx1_v)
    pltpu.sync_copy(idx2_hbm.at[pl.ds(base, n_per_w)], idx2_v)

    copies = []
    for k in range(0, n_per_w, _CHUNK):
        sl = pl.ds(k, _CHUNK)
        copies.append(pltpu.async_copy(
            emb_in_hbm.at[idx0_v.at[sl]], e0_v.at[sl], sem))
        copies.append(pltpu.async_copy(
            emb_in_hbm.at[idx1_v.at[sl]], e1_v.at[sl], sem))
        copies.append(pltpu.async_copy(
            emb_w_hbm.at[idx2_v.at[sl]], w_v.at[sl], sem))
        copies.append(pltpu.async_copy(
            emb_b_hbm.at[idx2_v.at[sl]], b_v.at[sl], sem))
    for c in copies:
        c.wait()

    lanes = lax.iota(jnp.int32, _L)

    def group(g, _):
        rows = g * _L + lanes
        acc = jnp.zeros((_L,), jnp.float32)
        for d in range(D):
            col = jnp.full((_L,), d, jnp.int32)
            a0 = plsc.load_gather(e0_v, [rows, col])
            a1 = plsc.load_gather(e1_v, [rows, col])
            aw = plsc.load_gather(w_v, [rows, col])
            acc = acc + (a0 + a1) * aw
        logit = acc * 0.5 + b_v[pl.ds(g * _L, _L)]
        out_v[pl.ds(g * _L, _L)] = 1.0 / (1.0 + jnp.exp(-logit))
        return _

    lax.fori_loop(0, n_per_w // _L, group, None)

    pltpu.sync_copy(out_v, out_hbm.at[pl.ds(base, n_per_w)])


def kernel(x, emb_in, emb_out_w, emb_out_b):
    B = x.shape[0]
    V, D = emb_in.shape
    n_per_w = B // _NW

    idx0 = x[:, 0]
    idx1 = x[:, 1]
    idx2 = x[:, 2]
    b_flat = emb_out_b.reshape(V)

    mesh = plsc.VectorSubcoreMesh(core_axis_name="c", subcore_axis_name="s")
    run = pl.kernel(
        functools.partial(_cbow_body, n_per_w, D),
        out_type=jax.ShapeDtypeStruct((B,), jnp.float32),
        mesh=mesh,
        scratch_types=[
            pltpu.VMEM((n_per_w,), jnp.int32),
            pltpu.VMEM((n_per_w,), jnp.int32),
            pltpu.VMEM((n_per_w,), jnp.int32),
            pltpu.VMEM((n_per_w, D), jnp.float32),
            pltpu.VMEM((n_per_w, D), jnp.float32),
            pltpu.VMEM((n_per_w, D), jnp.float32),
            pltpu.VMEM((n_per_w,), jnp.float32),
            pltpu.VMEM((n_per_w,), jnp.float32),
            pltpu.SemaphoreType.DMA,
        ],
        compiler_params=pltpu.CompilerParams(
            needs_layout_passes=False, use_tc_tiling_on_sc=False),
    )
    out = run(idx0, idx1, idx2, emb_in, emb_out_w, b_flat)
    return out.reshape(B, 1)
